# baseline (speedup 1.0000x reference)
import functools

import jax
import jax.numpy as jnp
from jax import lax
from jax.experimental import pallas as pl
from jax.experimental.pallas import tpu as pltpu
from jax.experimental.pallas import tpu_sc as plsc


def _table_transform_body(emb_ref, w1_ref, b1_ref, w2_ref, b2_ref, out_ref):
    h = jnp.dot(emb_ref[...], w1_ref[...], preferred_element_type=jnp.float32)
    h = jnp.maximum(h + b1_ref[...][None, :], 0.0)
    h = jnp.dot(h, w2_ref[...], preferred_element_type=jnp.float32)
    h = jnp.maximum(h + b2_ref[...][None, :], 0.0)
    out_ref[...] = h


def _table_transform(emb_table, w1, b1, w2, b2):
    v, d = emb_table.shape[0], w2.shape[1]
    return pl.pallas_call(
        _table_transform_body,
        out_shape=jax.ShapeDtypeStruct((v, d), jnp.float32),
    )(emb_table, w1, b1, w2, b2)


def _sum_partials_body(p_ref, o_ref):
    o_ref[...] = p_ref[0] + p_ref[1]


def _sum_partials(partials):
    _, g, d = partials.shape
    return pl.pallas_call(
        _sum_partials_body,
        out_shape=jax.ShapeDtypeStruct((g, d), jnp.float32),
    )(partials)


_CHUNK = 128


_K = 4


@functools.lru_cache(maxsize=None)
def _make_sc_kernel(n, v, d, g):
    info = plsc.get_sparse_core_info()
    nc, ns = info.num_cores, info.num_subcores
    nw = nc * ns

    full = n // _CHUNK
    rem = n - full * _CHUNK
    rem_owner = nw - 1
    base_trips = full // nw
    extra = full - base_trips * nw
    rows_per_tile = g // ns
    n_super = base_trips // _K
    tail_lo = n_super * _K
    assert g % ns == 0 and d % 16 == 0 and rem % 8 == 0

    mesh = plsc.VectorSubcoreMesh(core_axis_name="c", subcore_axis_name="s")

    scratch = [
        pltpu.VMEM((base_trips + 1, _CHUNK), jnp.int32),
        pltpu.VMEM((base_trips + 1, _CHUNK), jnp.int32),
        pltpu.VMEM((_K, _CHUNK, d), jnp.float32),
        pltpu.VMEM((max(rem, 8),), jnp.int32),
        pltpu.VMEM((max(rem, 8),), jnp.int32),
        pltpu.VMEM((max(rem, 8), d), jnp.float32),
        pltpu.VMEM((rows_per_tile, d), jnp.float32),
        pltpu.VMEM((v // ns, d), jnp.float32),
        pltpu.VMEM_SHARED((g, d), jnp.float32),
        pltpu.VMEM_SHARED((v, d), jnp.float32),
        pltpu.SemaphoreType.DMA,
        [pltpu.SemaphoreType.DMA] * _K,
        [pltpu.SemaphoreType.DMA] * _K,
        [pltpu.SemaphoreType.DMA] * _K,
    ]

    @functools.partial(
        pl.kernel,
        out_type=[
            jax.ShapeDtypeStruct((n, d), jnp.float32),
            jax.ShapeDtypeStruct((nc, g, d), jnp.float32),
        ],
        mesh=mesh,
        scratch_types=scratch,
    )
    def sc_kernel(r2_hbm, ut_hbm, n2g_hbm, out_hbm, part_hbm,
                  idx_blk, g_blk, rows, idx_r, g_r, rows_r, zeros_v,
                  r2_stage, acc, r2_sp, sem_i, sem_g, sem_st, sem_sc):
        cid = lax.axis_index("c")
        sid = lax.axis_index("s")
        wid = sid * nc + cid
        has_extra = wid < extra
        lo = wid * base_trips + jnp.minimum(wid, extra)

        def chunk_base(r):
            return pl.multiple_of((lo + r) * _CHUNK, 8)

        @pl.loop(0, base_trips)
        def _(r):
            pltpu.async_copy(ut_hbm.at[pl.ds(chunk_base(r), _CHUNK)],
                             idx_blk.at[r], sem_i)
            pltpu.async_copy(n2g_hbm.at[pl.ds(chunk_base(r), _CHUNK)],
                             g_blk.at[r], sem_i)

        @pl.when(has_extra)
        def _():
            pltpu.async_copy(ut_hbm.at[pl.ds(chunk_base(base_trips), _CHUNK)],
                             idx_blk.at[base_trips], sem_i)
            pltpu.async_copy(n2g_hbm.at[pl.ds(chunk_base(base_trips), _CHUNK)],
                             g_blk.at[base_trips], sem_i)

        @pl.loop(0, rows_per_tile)
        def _(r):
            for c0 in range(d // 16):
                zeros_v[r, pl.ds(c0 * 16, 16)] = jnp.zeros((16,), jnp.float32)

        pltpu.sync_copy(zeros_v, acc.at[pl.ds(sid * rows_per_tile, rows_per_tile)])

        tv = v // ns
        pltpu.sync_copy(r2_hbm.at[pl.ds(sid * tv, tv)], r2_stage)
        pltpu.sync_copy(r2_stage, r2_sp.at[pl.ds(sid * tv, tv)])
        plsc.subcore_barrier()

        @pl.loop(0, base_trips)
        def _(r):
            pltpu.make_async_copy(ut_hbm.at[pl.ds(chunk_base(r), _CHUNK)],
                                  idx_blk.at[r], sem_i).wait()
            pltpu.make_async_copy(n2g_hbm.at[pl.ds(chunk_base(r), _CHUNK)],
                                  g_blk.at[r], sem_i).wait()

        @pl.when(has_extra)
        def _():
            pltpu.make_async_copy(ut_hbm.at[pl.ds(chunk_base(base_trips), _CHUNK)],
                                  idx_blk.at[base_trips], sem_i).wait()
            pltpu.make_async_copy(n2g_hbm.at[pl.ds(chunk_base(base_trips), _CHUNK)],
                                  g_blk.at[base_trips], sem_i).wait()

        def gather(r, u):
            return pltpu.async_copy(r2_sp.at[idx_blk.at[r]], rows.at[u],
                                    sem_g[u])

        def store_scatter(r, u):
            st = pltpu.async_copy(rows.at[u],
                                  out_hbm.at[pl.ds(chunk_base(r), _CHUNK)],
                                  sem_st[u])
            sc = pltpu.async_copy(rows.at[u], acc.at[g_blk.at[r]], sem_sc[u],
                                  add=True)
            return st, sc

        def wait_store_scatter(r, u):
            pltpu.make_async_copy(rows.at[u],
                                  out_hbm.at[pl.ds(chunk_base(r), _CHUNK)],
                                  sem_st[u]).wait()
            pltpu.make_async_copy(rows.at[u], acc.at[g_blk.at[r]],
                                  sem_sc[u]).wait()

        @pl.loop(0, n_super)
        def _(p):
            r0 = p * _K
            gd = []
            for u in range(_K):
                @pl.when(p > 0)
                def _(u=u):
                    wait_store_scatter(r0 + u - _K, u)

                gd.append(gather(r0 + u, u))
            for u in range(_K):
                gd[u].wait()
                store_scatter(r0 + u, u)

        if n_super > 0:
            for u in range(_K):
                wait_store_scatter((n_super - 1) * _K + u, u)

        for r in range(tail_lo, base_trips):
            u = r - tail_lo
            gd = gather(r, u)
            gd.wait()
            st, sc = store_scatter(r, u)
            st.wait()
            sc.wait()

        @pl.when(has_extra)
        def _():
            gd = gather(base_trips, 0)
            gd.wait()
            st, sc = store_scatter(base_trips, 0)
            st.wait()
            sc.wait()

        if rem:
            @pl.when(wid == rem_owner)
            def _():
                base = full * _CHUNK
                pltpu.sync_copy(ut_hbm.at[pl.ds(base, rem)], idx_r)
                pltpu.sync_copy(n2g_hbm.at[pl.ds(base, rem)], g_r)
                pltpu.sync_copy(r2_sp.at[idx_r], rows_r)
                pltpu.sync_copy(rows_r, out_hbm.at[pl.ds(base, rem)])
                pltpu.sync_copy(rows_r, acc.at[g_r], add=True)

        plsc.subcore_barrier()
        pltpu.sync_copy(
            acc.at[pl.ds(sid * rows_per_tile, rows_per_tile)],
            part_hbm.at[cid, pl.ds(sid * rows_per_tile, rows_per_tile)],
        )

    return sc_kernel


def kernel(input, unit_type, node2graph, emb_table, W1, b1, W2, b2):
    del input
    n = unit_type.shape[0]
    d = W2.shape[1]
    v = emb_table.shape[0]
    g = 256

    r2 = _table_transform(emb_table, W1, b1, W2, b2)
    sc = _make_sc_kernel(n, v, d, g)
    node_feature, partials = sc(
        r2, unit_type.astype(jnp.int32), node2graph.astype(jnp.int32)
    )
    graph_feature = _sum_partials(partials)
    return graph_feature, node_feature


# speedup vs baseline: 1.2681x; 1.2681x over previous
import functools

import jax
import jax.numpy as jnp
from jax import lax
from jax.experimental import pallas as pl
from jax.experimental.pallas import tpu as pltpu
from jax.experimental.pallas import tpu_sc as plsc


def _table_transform_body(emb_ref, w1_ref, b1_ref, w2_ref, b2_ref, out_ref):
    h = jnp.dot(emb_ref[...], w1_ref[...], preferred_element_type=jnp.float32)
    h = jnp.maximum(h + b1_ref[...][None, :], 0.0)
    h = jnp.dot(h, w2_ref[...], preferred_element_type=jnp.float32)
    h = jnp.maximum(h + b2_ref[...][None, :], 0.0)
    out_ref[...] = h


def _table_transform(emb_table, w1, b1, w2, b2):
    v, d = emb_table.shape[0], w2.shape[1]
    return pl.pallas_call(
        _table_transform_body,
        out_shape=jax.ShapeDtypeStruct((v, d), jnp.float32),
    )(emb_table, w1, b1, w2, b2)


def _graph_readout_body(p_ref, r2_ref, o_ref):
    counts = p_ref[0] + p_ref[1]
    o_ref[...] = jnp.dot(counts, r2_ref[...],
                         preferred_element_type=jnp.float32)


def _graph_readout(partials, r2):
    _, g, v = partials.shape
    d = r2.shape[1]
    return pl.pallas_call(
        _graph_readout_body,
        out_shape=jax.ShapeDtypeStruct((g, d), jnp.float32),
    )(partials, r2)


_CHUNK = 128


_K = 4


@functools.lru_cache(maxsize=None)
def _make_sc_kernel(n, v, d, g):
    info = plsc.get_sparse_core_info()
    nc, ns = info.num_cores, info.num_subcores
    nw = nc * ns

    full = n // _CHUNK
    rem = n - full * _CHUNK
    rem_owner = nw - 1
    base_trips = full // nw
    extra = full - base_trips * nw
    gv = g * v
    zc = gv // ns
    n_super = base_trips // _K
    tail_lo = n_super * _K
    assert gv % (16 * ns) == 0 and d % 16 == 0 and rem % 16 == 0

    mesh = plsc.VectorSubcoreMesh(core_axis_name="c", subcore_axis_name="s")

    scratch = [
        pltpu.VMEM((base_trips + 1, _CHUNK), jnp.int32),
        pltpu.VMEM((base_trips + 1, _CHUNK), jnp.int32),
        pltpu.VMEM((_K, _CHUNK, d), jnp.float32),
        pltpu.VMEM((max(rem, 8),), jnp.int32),
        pltpu.VMEM((max(rem, 8),), jnp.int32),
        pltpu.VMEM((max(rem, 8), d), jnp.float32),
        pltpu.VMEM((max(rem, 16),), jnp.int32),
        pltpu.VMEM((_K, _CHUNK), jnp.int32),
        pltpu.VMEM((_CHUNK,), jnp.float32),
        pltpu.VMEM((zc,), jnp.float32),
        pltpu.VMEM((v // ns, d), jnp.float32),
        pltpu.VMEM_SHARED((gv,), jnp.float32),
        pltpu.VMEM_SHARED((v, d), jnp.float32),
        pltpu.SemaphoreType.DMA,
        [pltpu.SemaphoreType.DMA] * _K,
        [pltpu.SemaphoreType.DMA] * _K,
        [pltpu.SemaphoreType.DMA] * _K,
    ]

    @functools.partial(
        pl.kernel,
        out_type=[
            jax.ShapeDtypeStruct((n, d), jnp.float32),
            jax.ShapeDtypeStruct((nc, gv), jnp.float32),
        ],
        mesh=mesh,
        scratch_types=scratch,
    )
    def sc_kernel(r2_hbm, ut_hbm, n2g_hbm, out_hbm, part_hbm,
                  idx_blk, g_blk, rows, idx_r, g_r, rows_r, pidx_r, pidx,
                  ones_v, zeros_v, r2_stage, acc, r2_sp,
                  sem_i, sem_g, sem_st, sem_sc):
        cid = lax.axis_index("c")
        sid = lax.axis_index("s")
        wid = sid * nc + cid
        has_extra = wid < extra
        lo = wid * base_trips + jnp.minimum(wid, extra)

        def chunk_base(r):
            return pl.multiple_of((lo + r) * _CHUNK, 8)

        @pl.loop(0, base_trips)
        def _(r):
            pltpu.async_copy(ut_hbm.at[pl.ds(chunk_base(r), _CHUNK)],
                             idx_blk.at[r], sem_i)
            pltpu.async_copy(n2g_hbm.at[pl.ds(chunk_base(r), _CHUNK)],
                             g_blk.at[r], sem_i)

        @pl.when(has_extra)
        def _():
            pltpu.async_copy(ut_hbm.at[pl.ds(chunk_base(base_trips), _CHUNK)],
                             idx_blk.at[base_trips], sem_i)
            pltpu.async_copy(n2g_hbm.at[pl.ds(chunk_base(base_trips), _CHUNK)],
                             g_blk.at[base_trips], sem_i)

        @pl.loop(0, zc // 16)
        def _(i):
            zeros_v[pl.ds(i * 16, 16)] = jnp.zeros((16,), jnp.float32)

        @pl.loop(0, _CHUNK // 16)
        def _(i):
            ones_v[pl.ds(i * 16, 16)] = jnp.ones((16,), jnp.float32)

        pltpu.sync_copy(zeros_v, acc.at[pl.ds(sid * zc, zc)])

        tv = v // ns
        pltpu.sync_copy(r2_hbm.at[pl.ds(sid * tv, tv)], r2_stage)
        pltpu.sync_copy(r2_stage, r2_sp.at[pl.ds(sid * tv, tv)])
        plsc.subcore_barrier()

        @pl.loop(0, base_trips)
        def _(r):
            pltpu.make_async_copy(ut_hbm.at[pl.ds(chunk_base(r), _CHUNK)],
                                  idx_blk.at[r], sem_i).wait()
            pltpu.make_async_copy(n2g_hbm.at[pl.ds(chunk_base(r), _CHUNK)],
                                  g_blk.at[r], sem_i).wait()

        @pl.when(has_extra)
        def _():
            pltpu.make_async_copy(ut_hbm.at[pl.ds(chunk_base(base_trips), _CHUNK)],
                                  idx_blk.at[base_trips], sem_i).wait()
            pltpu.make_async_copy(n2g_hbm.at[pl.ds(chunk_base(base_trips), _CHUNK)],
                                  g_blk.at[base_trips], sem_i).wait()

        def gather(r, u):
            return pltpu.async_copy(r2_sp.at[idx_blk.at[r]], rows.at[u],
                                    sem_g[u])

        def store_scatter(r, u):
            st = pltpu.async_copy(rows.at[u],
                                  out_hbm.at[pl.ds(chunk_base(r), _CHUNK)],
                                  sem_st[u])
            for c0 in range(_CHUNK // 16):
                s = pl.ds(c0 * 16, 16)
                pidx[u, s] = g_blk[r, s] * v + idx_blk[r, s]
            sc = pltpu.async_copy(ones_v, acc.at[pidx.at[u]], sem_sc[u],
                                  add=True)
            return st, sc

        def wait_store_scatter(r, u):
            pltpu.make_async_copy(rows.at[u],
                                  out_hbm.at[pl.ds(chunk_base(r), _CHUNK)],
                                  sem_st[u]).wait()
            pltpu.make_async_copy(ones_v, acc.at[pidx.at[u]],
                                  sem_sc[u]).wait()

        @pl.loop(0, n_super)
        def _(p):
            r0 = p * _K
            gd = []
            for u in range(_K):
                @pl.when(p > 0)
                def _(u=u):
                    wait_store_scatter(r0 + u - _K, u)

                gd.append(gather(r0 + u, u))
            for u in range(_K):
                gd[u].wait()
                store_scatter(r0 + u, u)

        if n_super > 0:
            for u in range(_K):
                wait_store_scatter((n_super - 1) * _K + u, u)

        for r in range(tail_lo, base_trips):
            u = r - tail_lo
            gd = gather(r, u)
            gd.wait()
            st, sc = store_scatter(r, u)
            st.wait()
            sc.wait()

        @pl.when(has_extra)
        def _():
            gd = gather(base_trips, 0)
            gd.wait()
            st, sc = store_scatter(base_trips, 0)
            st.wait()
            sc.wait()

        if rem:
            @pl.when(wid == rem_owner)
            def _():
                base = full * _CHUNK
                pltpu.sync_copy(ut_hbm.at[pl.ds(base, rem)], idx_r)
                pltpu.sync_copy(n2g_hbm.at[pl.ds(base, rem)], g_r)
                pltpu.sync_copy(r2_sp.at[idx_r], rows_r)
                pltpu.sync_copy(rows_r, out_hbm.at[pl.ds(base, rem)])
                for c0 in range(rem // 16):
                    s = pl.ds(c0 * 16, 16)
                    pidx_r[s] = g_r[s] * v + idx_r[s]
                pltpu.sync_copy(ones_v.at[pl.ds(0, rem)], acc.at[pidx_r],
                                add=True)

        plsc.subcore_barrier()
        pltpu.sync_copy(
            acc.at[pl.ds(sid * zc, zc)],
            part_hbm.at[cid, pl.ds(sid * zc, zc)],
        )

    return sc_kernel


def kernel(input, unit_type, node2graph, emb_table, W1, b1, W2, b2):
    del input
    n = unit_type.shape[0]
    d = W2.shape[1]
    v = emb_table.shape[0]
    g = 256

    r2 = _table_transform(emb_table, W1, b1, W2, b2)
    sc = _make_sc_kernel(n, v, d, g)
    node_feature, counts = sc(
        r2, unit_type.astype(jnp.int32), node2graph.astype(jnp.int32)
    )
    graph_feature = _graph_readout(counts.reshape(counts.shape[0], g, v), r2)
    return graph_feature, node_feature


# speedup vs baseline: 1.3128x; 1.0352x over previous
import functools

import jax
import jax.numpy as jnp
from jax import lax
from jax.experimental import pallas as pl
from jax.experimental.pallas import tpu as pltpu
from jax.experimental.pallas import tpu_sc as plsc


def _table_transform_body(emb_ref, w1_ref, b1_ref, w2_ref, b2_ref, out_ref):
    h = jnp.dot(emb_ref[...], w1_ref[...], preferred_element_type=jnp.float32)
    h = jnp.maximum(h + b1_ref[...][None, :], 0.0)
    h = jnp.dot(h, w2_ref[...], preferred_element_type=jnp.float32)
    h = jnp.maximum(h + b2_ref[...][None, :], 0.0)
    out_ref[...] = h


def _table_transform(emb_table, w1, b1, w2, b2):
    v, d = emb_table.shape[0], w2.shape[1]
    return pl.pallas_call(
        _table_transform_body,
        out_shape=jax.ShapeDtypeStruct((v, d), jnp.float32),
    )(emb_table, w1, b1, w2, b2)


def _graph_readout_body(p_ref, r2_ref, o_ref):
    counts = p_ref[0] + p_ref[1]
    o_ref[...] = jnp.dot(counts, r2_ref[...],
                         preferred_element_type=jnp.float32)


def _graph_readout(partials, r2):
    _, g, v = partials.shape
    d = r2.shape[1]
    return pl.pallas_call(
        _graph_readout_body,
        out_shape=jax.ShapeDtypeStruct((g, d), jnp.float32),
    )(partials, r2)


_CHUNK = 128


_K = 4


@functools.lru_cache(maxsize=None)
def _make_sc_kernel(n, v, d, g):
    info = plsc.get_sparse_core_info()
    nc, ns = info.num_cores, info.num_subcores
    nw = nc * ns

    full = n // _CHUNK
    rem = n - full * _CHUNK
    rem_owner = nw - 1
    base_trips = full // nw
    extra = full - base_trips * nw
    gv = g * v
    zc = gv // ns
    n_super = base_trips // _K
    tail_lo = n_super * _K
    assert gv % (16 * ns) == 0 and d % 16 == 0 and rem % 16 == 0

    mesh = plsc.VectorSubcoreMesh(core_axis_name="c", subcore_axis_name="s")

    scratch = [
        pltpu.VMEM((base_trips + 1, _CHUNK), jnp.int32),
        pltpu.VMEM((base_trips + 1, _CHUNK), jnp.int32),
        pltpu.VMEM((_K, _CHUNK, d), jnp.float32),
        pltpu.VMEM((max(rem, 8),), jnp.int32),
        pltpu.VMEM((max(rem, 8),), jnp.int32),
        pltpu.VMEM((max(rem, 8), d), jnp.float32),
        pltpu.VMEM((max(rem, 16),), jnp.int32),
        pltpu.VMEM((_K, _CHUNK), jnp.int32),
        pltpu.VMEM((_CHUNK,), jnp.float32),
        pltpu.VMEM((zc,), jnp.float32),
        pltpu.VMEM((v // ns, d), jnp.float32),
        pltpu.VMEM_SHARED((gv,), jnp.float32),
        pltpu.VMEM_SHARED((v, d), jnp.float32),
        pltpu.SemaphoreType.DMA,
        [pltpu.SemaphoreType.DMA] * _K,
        [pltpu.SemaphoreType.DMA] * _K,
        [pltpu.SemaphoreType.DMA] * _K,
    ]

    @functools.partial(
        pl.kernel,
        out_type=[
            jax.ShapeDtypeStruct((n, d), jnp.float32),
            jax.ShapeDtypeStruct((nc, g, v), jnp.float32),
        ],
        mesh=mesh,
        scratch_types=scratch,
    )
    def sc_kernel(r2_hbm, ut_hbm, n2g_hbm, out_hbm, part_hbm,
                  idx_blk, g_blk, rows, idx_r, g_r, rows_r, pidx_r, pidx,
                  ones_v, zeros_v, r2_stage, acc, r2_sp,
                  sem_i, sem_g, sem_st, sem_sc):
        cid = lax.axis_index("c")
        sid = lax.axis_index("s")
        wid = sid * nc + cid
        has_extra = wid < extra
        lo = wid * base_trips + jnp.minimum(wid, extra)

        def chunk_base(r):
            return pl.multiple_of((lo + r) * _CHUNK, 8)

        @pl.loop(0, base_trips)
        def _(r):
            pltpu.async_copy(ut_hbm.at[pl.ds(chunk_base(r), _CHUNK)],
                             idx_blk.at[r], sem_i)
            pltpu.async_copy(n2g_hbm.at[pl.ds(chunk_base(r), _CHUNK)],
                             g_blk.at[r], sem_i)

        @pl.when(has_extra)
        def _():
            pltpu.async_copy(ut_hbm.at[pl.ds(chunk_base(base_trips), _CHUNK)],
                             idx_blk.at[base_trips], sem_i)
            pltpu.async_copy(n2g_hbm.at[pl.ds(chunk_base(base_trips), _CHUNK)],
                             g_blk.at[base_trips], sem_i)

        @pl.loop(0, zc // 16)
        def _(i):
            zeros_v[pl.ds(i * 16, 16)] = jnp.zeros((16,), jnp.float32)

        @pl.loop(0, _CHUNK // 16)
        def _(i):
            ones_v[pl.ds(i * 16, 16)] = jnp.ones((16,), jnp.float32)

        pltpu.sync_copy(zeros_v, acc.at[pl.ds(sid * zc, zc)])

        tv = v // ns
        pltpu.sync_copy(r2_hbm.at[pl.ds(sid * tv, tv)], r2_stage)
        pltpu.sync_copy(r2_stage, r2_sp.at[pl.ds(sid * tv, tv)])
        plsc.subcore_barrier()

        @pl.loop(0, base_trips)
        def _(r):
            pltpu.make_async_copy(ut_hbm.at[pl.ds(chunk_base(r), _CHUNK)],
                                  idx_blk.at[r], sem_i).wait()
            pltpu.make_async_copy(n2g_hbm.at[pl.ds(chunk_base(r), _CHUNK)],
                                  g_blk.at[r], sem_i).wait()

        @pl.when(has_extra)
        def _():
            pltpu.make_async_copy(ut_hbm.at[pl.ds(chunk_base(base_trips), _CHUNK)],
                                  idx_blk.at[base_trips], sem_i).wait()
            pltpu.make_async_copy(n2g_hbm.at[pl.ds(chunk_base(base_trips), _CHUNK)],
                                  g_blk.at[base_trips], sem_i).wait()

        def gather(r, u):
            return pltpu.async_copy(r2_sp.at[idx_blk.at[r]], rows.at[u],
                                    sem_g[u])

        def store_scatter(r, u):
            st = pltpu.async_copy(rows.at[u],
                                  out_hbm.at[pl.ds(chunk_base(r), _CHUNK)],
                                  sem_st[u])
            for c0 in range(_CHUNK // 16):
                s = pl.ds(c0 * 16, 16)
                pidx[u, s] = g_blk[r, s] * v + idx_blk[r, s]
            sc = pltpu.async_copy(ones_v, acc.at[pidx.at[u]], sem_sc[u],
                                  add=True)
            return st, sc

        def wait_store_scatter(r, u):
            pltpu.make_async_copy(rows.at[u],
                                  out_hbm.at[pl.ds(chunk_base(r), _CHUNK)],
                                  sem_st[u]).wait()
            pltpu.make_async_copy(ones_v, acc.at[pidx.at[u]],
                                  sem_sc[u]).wait()

        @pl.loop(0, n_super)
        def _(p):
            r0 = p * _K
            gd = []
            for u in range(_K):
                @pl.when(p > 0)
                def _(u=u):
                    wait_store_scatter(r0 + u - _K, u)

                gd.append(gather(r0 + u, u))
            for u in range(_K):
                gd[u].wait()
                store_scatter(r0 + u, u)

        if n_super > 0:
            for u in range(_K):
                wait_store_scatter((n_super - 1) * _K + u, u)

        for r in range(tail_lo, base_trips):
            u = r - tail_lo
            gd = gather(r, u)
            gd.wait()
            st, sc = store_scatter(r, u)
            st.wait()
            sc.wait()

        @pl.when(has_extra)
        def _():
            gd = gather(base_trips, 0)
            gd.wait()
            st, sc = store_scatter(base_trips, 0)
            st.wait()
            sc.wait()

        if rem:
            @pl.when(wid == rem_owner)
            def _():
                base = full * _CHUNK
                pltpu.sync_copy(ut_hbm.at[pl.ds(base, rem)], idx_r)
                pltpu.sync_copy(n2g_hbm.at[pl.ds(base, rem)], g_r)
                pltpu.sync_copy(r2_sp.at[idx_r], rows_r)
                pltpu.sync_copy(rows_r, out_hbm.at[pl.ds(base, rem)])
                for c0 in range(rem // 16):
                    s = pl.ds(c0 * 16, 16)
                    pidx_r[s] = g_r[s] * v + idx_r[s]
                pltpu.sync_copy(ones_v.at[pl.ds(0, rem)], acc.at[pidx_r],
                                add=True)

        plsc.subcore_barrier()
        g_per_tile = g // ns
        for rr in range(g_per_tile):
            pltpu.async_copy(
                acc.at[pl.ds((sid * g_per_tile + rr) * v, v)],
                part_hbm.at[cid, sid * g_per_tile + rr], sem_i)
        for rr in range(g_per_tile):
            pltpu.make_async_copy(
                acc.at[pl.ds((sid * g_per_tile + rr) * v, v)],
                part_hbm.at[cid, sid * g_per_tile + rr], sem_i).wait()

    return sc_kernel


def kernel(input, unit_type, node2graph, emb_table, W1, b1, W2, b2):
    del input
    n = unit_type.shape[0]
    d = W2.shape[1]
    v = emb_table.shape[0]
    g = 256

    r2 = _table_transform(emb_table, W1, b1, W2, b2)
    sc = _make_sc_kernel(n, v, d, g)
    node_feature, counts = sc(
        r2, unit_type.astype(jnp.int32), node2graph.astype(jnp.int32)
    )
    graph_feature = _graph_readout(counts, r2)
    return graph_feature, node_feature
